# baseline (speedup 1.0000x reference)
import jax
import jax.numpy as jnp
from jax.experimental import pallas as pl

_VOCAB = 1000000
_EMB = 64
_B = 4096

_CH = 2097152
_N = _VOCAB * _EMB
_GRID = -(-_N // _CH)


def _probe_body(t_ref, o_ref):
    o_ref[...] = t_ref[pl.ds(0, 1024)]


def kernel(x, table, W, b):
    tf = table.reshape(_N)
    out = pl.pallas_call(
        _probe_body,
        grid=(_GRID,),
        in_specs=[pl.BlockSpec((_CH,), lambda i: (i,))],
        out_specs=pl.BlockSpec((1024,), lambda i: (i,)),
        out_shape=jax.ShapeDtypeStruct((_GRID * 1024,), jnp.float32),
    )(tf)
    return out[:_B].reshape(_B, 1)


# speedup vs baseline: 1.2114x; 1.2114x over previous
import functools

import jax
import jax.numpy as jnp
from jax import lax
from jax.experimental import pallas as pl
from jax.experimental.pallas import tpu as pltpu
from jax.experimental.pallas import tpu_sc as plsc

_VOCAB = 1000000
_EMB = 64
_B = 4096
_L = 200

_CH = 8192
_NBUF = 4
_STEP_ROWS = _CH * _NBUF
_NG = -(-_VOCAB // _STEP_ROWS)
_NCHUNK = _NG * _NBUF
_TVP = _NCHUNK * _CH
_CLAMP = _VOCAB - _CH
_TAIL = (_NCHUNK - 2) * _CH
_SHIFT = _TAIL - _CLAMP

_NC, _NS = 2, 16
_NW = _NC * _NS
_DPW = _B // _NW
_IPW = _L * _DPW


def _tv_body(w_ref, tbl_ref, o_ref, b0, b1, b2, b3, s0, s1, s2, s3):
    bufs = (b0, b1, b2, b3)
    sems = (s0, s1, s2, s3)
    g = pl.program_id(0)
    wv = w_ref[...]

    def cbase(c):
        return jnp.minimum(c * _CH, _CLAMP)

    def issue(c, k):
        pltpu.async_copy(tbl_ref.at[pl.ds(cbase(c), _CH), :], bufs[k],
                         sems[k])

    @pl.when(g == 0)
    def _():
        for k in range(_NBUF):
            issue(k, k)

    for k in range(_NBUF):
        c = g * _NBUF + k
        cs = cbase(c)
        pltpu.make_async_copy(tbl_ref.at[pl.ds(cs, _CH), :], bufs[k],
                              sems[k]).wait()
        dot = jnp.sum(bufs[k][...] * wv, axis=1)
        gid = lax.broadcasted_iota(jnp.int32, (_CH,), 0) + cs
        o_ref[pl.ds(k * _CH, _CH)] = jnp.where(gid == 0, 0.0, dot)
        nxt = c + _NBUF

        @pl.when(nxt < _NCHUNK)
        def _():
            issue(nxt, k)


def _compute_tv(table, W):
    return pl.pallas_call(
        _tv_body,
        grid=(_NG,),
        in_specs=[
            pl.BlockSpec((1, _EMB), lambda i: (0, 0)),
            pl.BlockSpec(memory_space=pltpu.MemorySpace.HBM),
        ],
        out_specs=pl.BlockSpec((_STEP_ROWS,), lambda i: (i,)),
        out_shape=jax.ShapeDtypeStruct((_TVP,), jnp.float32),
        scratch_shapes=(
            [pltpu.VMEM((_CH, _EMB), jnp.float32)] * 4
            + [pltpu.SemaphoreType.DMA] * 4
        ),
    )(W, table)


def _xpose_body(x_ref, o_ref):
    xt = x_ref[...].T
    o_ref[...] = xt + jnp.where(xt >= _TAIL, _SHIFT, 0)


def _transpose_x(x):
    return pl.pallas_call(
        _xpose_body,
        grid=(_NW,),
        in_specs=[pl.BlockSpec((_DPW, _L), lambda w: (w, 0))],
        out_specs=pl.BlockSpec((_L, _DPW), lambda w: (w, 0)),
        out_shape=jax.ShapeDtypeStruct((_NW * _L, _DPW), jnp.int32),
    )(x)


@functools.partial(
    pl.kernel,
    out_type=jax.ShapeDtypeStruct((_B,), jnp.float32),
    mesh=plsc.VectorSubcoreMesh(core_axis_name="c", subcore_axis_name="s"),
    scratch_types=[
        pltpu.VMEM((_IPW,), jnp.int32),
        pltpu.VMEM((_IPW,), jnp.float32),
        pltpu.VMEM((_DPW,), jnp.float32),
        pltpu.VMEM((16,), jnp.float32),
        pltpu.SemaphoreType.DMA,
    ],
)
def _sc_pool(xw_hbm, tv_hbm, b_hbm, out_hbm, idx_v, vals_v, out_v, b_v, sem):
    wid = lax.axis_index("s") * _NC + lax.axis_index("c")
    base = wid * _DPW
    pltpu.sync_copy(b_hbm, b_v)
    pltpu.sync_copy(xw_hbm.at[pl.ds(wid * _IPW, _IPW)], idx_v)
    pltpu.async_copy(tv_hbm.at[idx_v], vals_v, sem).wait()
    bias = b_v[...]
    nchunk = _DPW // 16

    def body(r, accs):
        off = r * _DPW
        return tuple(a + vals_v[pl.ds(off + 16 * c, 16)]
                     for c, a in enumerate(accs))

    accs = lax.fori_loop(0, _L, body, (bias,) * nchunk)
    for c in range(nchunk):
        out_v[pl.ds(c * 16, 16)] = accs[c]
    pltpu.sync_copy(out_v, out_hbm.at[pl.ds(base, _DPW)])


def kernel(x, table, W, b):
    xw = _transpose_x(x.astype(jnp.int32)).reshape(_B * _L)
    b16 = jnp.broadcast_to(b.astype(jnp.float32), (16,))
    ws = W.astype(jnp.float32) * (1.0 / _L)
    tv = _compute_tv(table, ws)
    out = _sc_pool(xw, tv, b16)
    return out.reshape(_B, 1)


# speedup vs baseline: 1.2157x; 1.0036x over previous
import functools

import jax
import jax.numpy as jnp
from jax import lax
from jax.experimental import pallas as pl
from jax.experimental.pallas import tpu as pltpu
from jax.experimental.pallas import tpu_sc as plsc

_VOCAB = 1000000
_EMB = 64
_B = 4096
_L = 200

_CH = 4096
_NBUF = 8
_STEP_ROWS = _CH * _NBUF
_NG = -(-_VOCAB // _STEP_ROWS)
_NCHUNK = _NG * _NBUF
_TVP = _NCHUNK * _CH
_CLAMP = _VOCAB - _CH
_TAIL = (_CLAMP // _CH + 1) * _CH
_SHIFT = _TAIL - _CLAMP

_NC, _NS = 2, 16
_NW = _NC * _NS
_DPW = _B // _NW
_IPW = _L * _DPW


def _tv_body(w_ref, tbl_ref, o_ref, *scratch):
    bufs = scratch[:_NBUF]
    sems = scratch[_NBUF:]
    g = pl.program_id(0)
    wv = w_ref[...]

    def cbase(c):
        return jnp.minimum(c * _CH, _CLAMP)

    def issue(c, k):
        pltpu.async_copy(tbl_ref.at[pl.ds(cbase(c), _CH), :], bufs[k],
                         sems[k])

    @pl.when(g == 0)
    def _():
        for k in range(_NBUF):
            issue(k, k)

    for k in range(_NBUF):
        c = g * _NBUF + k
        cs = cbase(c)
        pltpu.make_async_copy(tbl_ref.at[pl.ds(cs, _CH), :], bufs[k],
                              sems[k]).wait()
        dot = jnp.sum(bufs[k][...] * wv, axis=1)
        gid = lax.broadcasted_iota(jnp.int32, (_CH,), 0) + cs
        o_ref[pl.ds(k * _CH, _CH)] = jnp.where(gid == 0, 0.0, dot)
        nxt = c + _NBUF

        @pl.when(nxt < _NCHUNK)
        def _():
            issue(nxt, k)


def _compute_tv(table, W):
    return pl.pallas_call(
        _tv_body,
        grid=(_NG,),
        in_specs=[
            pl.BlockSpec((1, _EMB), lambda i: (0, 0)),
            pl.BlockSpec(memory_space=pltpu.MemorySpace.HBM),
        ],
        out_specs=pl.BlockSpec((_STEP_ROWS,), lambda i: (i,)),
        out_shape=jax.ShapeDtypeStruct((_TVP,), jnp.float32),
        scratch_shapes=(
            [pltpu.VMEM((_CH, _EMB), jnp.float32)] * _NBUF
            + [pltpu.SemaphoreType.DMA] * _NBUF
        ),
    )(W, table)


def _xpose_body(x_ref, o_ref):
    xt = x_ref[...].T
    o_ref[...] = xt + jnp.where(xt >= _TAIL, _SHIFT, 0)


def _transpose_x(x):
    return pl.pallas_call(
        _xpose_body,
        grid=(_NW,),
        in_specs=[pl.BlockSpec((_DPW, _L), lambda w: (w, 0))],
        out_specs=pl.BlockSpec((_L, _DPW), lambda w: (w, 0)),
        out_shape=jax.ShapeDtypeStruct((_NW * _L, _DPW), jnp.int32),
    )(x)


@functools.partial(
    pl.kernel,
    out_type=jax.ShapeDtypeStruct((_B,), jnp.float32),
    mesh=plsc.VectorSubcoreMesh(core_axis_name="c", subcore_axis_name="s"),
    scratch_types=[
        pltpu.VMEM((_IPW,), jnp.int32),
        pltpu.VMEM((_IPW,), jnp.float32),
        pltpu.VMEM((_DPW,), jnp.float32),
        pltpu.VMEM((16,), jnp.float32),
        pltpu.SemaphoreType.DMA,
    ],
)
def _sc_pool(xw_hbm, tv_hbm, b_hbm, out_hbm, idx_v, vals_v, out_v, b_v, sem):
    wid = lax.axis_index("s") * _NC + lax.axis_index("c")
    base = wid * _DPW
    pltpu.sync_copy(b_hbm, b_v)
    pltpu.sync_copy(xw_hbm.at[pl.ds(wid * _IPW, _IPW)], idx_v)
    pltpu.async_copy(tv_hbm.at[idx_v], vals_v, sem).wait()
    bias = b_v[...]
    nchunk = _DPW // 16

    def body(r, accs):
        off = r * _DPW
        return tuple(a + vals_v[pl.ds(off + 16 * c, 16)]
                     for c, a in enumerate(accs))

    accs = lax.fori_loop(0, _L, body, (bias,) * nchunk)
    for c in range(nchunk):
        out_v[pl.ds(c * 16, 16)] = accs[c]
    pltpu.sync_copy(out_v, out_hbm.at[pl.ds(base, _DPW)])


def kernel(x, table, W, b):
    xw = _transpose_x(x.astype(jnp.int32)).reshape(_B * _L)
    b16 = jnp.broadcast_to(b.astype(jnp.float32), (16,))
    ws = W.astype(jnp.float32) * (1.0 / _L)
    tv = _compute_tv(table, ws)
    out = _sc_pool(xw, tv, b16)
    return out.reshape(_B, 1)


# speedup vs baseline: 1.2393x; 1.0194x over previous
import functools

import jax
import jax.numpy as jnp
from jax import lax
from jax.experimental import pallas as pl
from jax.experimental.pallas import tpu as pltpu
from jax.experimental.pallas import tpu_sc as plsc

_VOCAB = 1000000
_EMB = 64
_B = 4096
_L = 200

_CH = 4096
_NBUF = 8
_STEP_ROWS = _CH * _NBUF
_NG = -(-_VOCAB // _STEP_ROWS)
_NCHUNK = _NG * _NBUF
_TVP = _NCHUNK * _CH
_CLAMP = _VOCAB - _CH
_TAIL = (_CLAMP // _CH + 1) * _CH
_SHIFT = _TAIL - _CLAMP

_NC, _NS = 2, 16
_NW = _NC * _NS
_DPW = _B // _NW
_IPW = _L * _DPW


def _tv_body(w_ref, x_ref, tbl_ref, o_ref, xw_ref, *scratch):
    bufs = scratch[:_NBUF]
    sems = scratch[_NBUF:]
    g = pl.program_id(0)
    wv = w_ref[...]

    xb = x_ref[...]
    xt = jnp.concatenate([xb[:_DPW].T, xb[_DPW:].T], axis=0)
    xw_ref[...] = xt + jnp.where(xt >= _TAIL, _SHIFT, 0)

    def cbase(c):
        return jnp.minimum(c * _CH, _CLAMP)

    def issue(c, k):
        pltpu.async_copy(tbl_ref.at[pl.ds(cbase(c), _CH), :], bufs[k],
                         sems[k])

    @pl.when(g == 0)
    def _():
        for k in range(_NBUF):
            issue(k, k)

    for k in range(_NBUF):
        c = g * _NBUF + k
        cs = cbase(c)
        pltpu.make_async_copy(tbl_ref.at[pl.ds(cs, _CH), :], bufs[k],
                              sems[k]).wait()
        dot = jnp.sum(bufs[k][...] * wv, axis=1)
        gid = lax.broadcasted_iota(jnp.int32, (_CH,), 0) + cs
        o_ref[pl.ds(k * _CH, _CH)] = jnp.where(gid == 0, 0.0, dot)
        nxt = c + _NBUF

        @pl.when(nxt < _NCHUNK)
        def _():
            issue(nxt, k)


def _compute_tv(table, W, x):
    return pl.pallas_call(
        _tv_body,
        grid=(_NG,),
        in_specs=[
            pl.BlockSpec((1, _EMB), lambda i: (0, 0)),
            pl.BlockSpec((2 * _DPW, _L),
                         lambda i: (jnp.minimum(i, _NW // 2 - 1), 0)),
            pl.BlockSpec(memory_space=pltpu.MemorySpace.HBM),
        ],
        out_specs=[
            pl.BlockSpec((_STEP_ROWS,), lambda i: (i,)),
            pl.BlockSpec((2 * _L, _DPW),
                         lambda i: (jnp.minimum(i, _NW // 2 - 1), 0)),
        ],
        out_shape=[
            jax.ShapeDtypeStruct((_TVP,), jnp.float32),
            jax.ShapeDtypeStruct((_NW * _L, _DPW), jnp.int32),
        ],
        scratch_shapes=(
            [pltpu.VMEM((_CH, _EMB), jnp.float32)] * _NBUF
            + [pltpu.SemaphoreType.DMA] * _NBUF
        ),
    )(W, x, table)


@functools.partial(
    pl.kernel,
    out_type=jax.ShapeDtypeStruct((_B,), jnp.float32),
    mesh=plsc.VectorSubcoreMesh(core_axis_name="c", subcore_axis_name="s"),
    scratch_types=[
        pltpu.VMEM((_IPW,), jnp.int32),
        pltpu.VMEM((_IPW,), jnp.float32),
        pltpu.VMEM((_DPW,), jnp.float32),
        pltpu.VMEM((16,), jnp.float32),
        pltpu.SemaphoreType.DMA,
    ],
)
def _sc_pool(xw_hbm, tv_hbm, b_hbm, out_hbm, idx_v, vals_v, out_v, b_v, sem):
    wid = lax.axis_index("s") * _NC + lax.axis_index("c")
    base = wid * _DPW
    pltpu.sync_copy(b_hbm, b_v)
    pltpu.sync_copy(xw_hbm.at[pl.ds(wid * _IPW, _IPW)], idx_v)
    pltpu.async_copy(tv_hbm.at[idx_v], vals_v, sem).wait()
    bias = b_v[...]
    nchunk = _DPW // 16

    def body(r, accs):
        off = r * _DPW
        return tuple(a + vals_v[pl.ds(off + 16 * c, 16)]
                     for c, a in enumerate(accs))

    accs = lax.fori_loop(0, _L, body, (bias,) * nchunk)
    for c in range(nchunk):
        out_v[pl.ds(c * 16, 16)] = accs[c]
    pltpu.sync_copy(out_v, out_hbm.at[pl.ds(base, _DPW)])


def kernel(x, table, W, b):
    b16 = jnp.broadcast_to(b.astype(jnp.float32), (16,))
    ws = W.astype(jnp.float32) * (1.0 / _L)
    tv, xw = _compute_tv(table, ws, x.astype(jnp.int32))
    out = _sc_pool(xw.reshape(_B * _L), tv, b16)
    return out.reshape(_B, 1)
